# Initial kernel scaffold; baseline (speedup 1.0000x reference)
import jax
import jax.numpy as jnp
from jax.experimental import pallas as pl


def kernel(x, edge_index, edge_attr, params):
    raise NotImplementedError("write your pallas kernel here")


# speedup vs baseline: 2.5722x; 2.5722x over previous
import functools

import jax
import jax.numpy as jnp
from jax import lax
from jax.experimental import pallas as pl
from jax.experimental.pallas import tpu as pltpu
from jax.experimental.pallas import tpu_sc as plsc

H = 128
G = 128
NC = 2
NS = 16
NW = NC * NS
NPAD = 10240


def _sc_mesh():
    return plsc.VectorSubcoreMesh(core_axis_name="c", subcore_axis_name="s")


def _sc_gather(x, dst, src):
    E = dst.shape[0]
    n_groups = E // G
    n_iters = (n_groups + NW - 1) // NW

    @functools.partial(
        pl.kernel,
        mesh=_sc_mesh(),
        out_type=(
            jax.ShapeDtypeStruct((E, H), jnp.float32),
            jax.ShapeDtypeStruct((E, H), jnp.float32),
        ),
        scratch_types=[
            pltpu.VMEM((G,), jnp.int32),
            pltpu.VMEM((G, H), jnp.float32),
            pltpu.VMEM((G,), jnp.int32),
            pltpu.VMEM((G, H), jnp.float32),
            pltpu.SemaphoreType.DMA,
            pltpu.SemaphoreType.DMA,
        ],
    )
    def k(x_hbm, dst_hbm, src_hbm, xi_hbm, xj_hbm,
          idx_d, rows_d, idx_s, rows_s, sem_d, sem_s):
        wid = lax.axis_index("s") * NC + lax.axis_index("c")

        def body(t, carry):
            g = wid + NW * t

            @pl.when(g < n_groups)
            def _():
                base = g * G
                pltpu.sync_copy(dst_hbm.at[pl.ds(base, G)], idx_d)
                pltpu.sync_copy(src_hbm.at[pl.ds(base, G)], idx_s)
                cp_d = pltpu.async_copy(x_hbm.at[idx_d], rows_d, sem_d)
                cp_s = pltpu.async_copy(x_hbm.at[idx_s], rows_s, sem_s)
                cp_d.wait()
                cp_s.wait()
                pltpu.sync_copy(rows_d, xi_hbm.at[pl.ds(base, G)])
                pltpu.sync_copy(rows_s, xj_hbm.at[pl.ds(base, G)])

            return carry

        lax.fori_loop(0, n_iters, body, 0)

    return k(x, dst, src)


def _sc_scatter(msg, dst, zeros):
    E = msg.shape[0]
    n_groups = E // G
    n_iters = (n_groups + NW - 1) // NW
    rpt = NPAD // NS

    @functools.partial(
        pl.kernel,
        mesh=_sc_mesh(),
        out_type=jax.ShapeDtypeStruct((NC, NPAD, H), jnp.float32),
        scratch_types=[
            pltpu.VMEM((G,), jnp.int32),
            pltpu.VMEM((G, H), jnp.float32),
            pltpu.VMEM_SHARED((NPAD, H), jnp.float32),
            pltpu.SemaphoreType.DMA,
        ],
    )
    def k(msg_hbm, dst_hbm, zeros_hbm, out_hbm, idx_v, rows_v, acc_sh, sem):
        cid = lax.axis_index("c")
        sid = lax.axis_index("s")
        wid = sid * NC + cid

        pltpu.sync_copy(zeros_hbm.at[pl.ds(sid * rpt, rpt)],
                        acc_sh.at[pl.ds(sid * rpt, rpt)])
        plsc.subcore_barrier()

        def body(t, carry):
            g = wid + NW * t

            @pl.when(g < n_groups)
            def _():
                base = g * G
                pltpu.sync_copy(dst_hbm.at[pl.ds(base, G)], idx_v)
                pltpu.sync_copy(msg_hbm.at[pl.ds(base, G)], rows_v)
                pltpu.sync_copy(rows_v, acc_sh.at[idx_v], add=True)

            return carry

        lax.fori_loop(0, n_iters, body, 0)
        plsc.subcore_barrier()
        pltpu.sync_copy(acc_sh.at[pl.ds(sid * rpt, rpt)],
                        out_hbm.at[cid, pl.ds(sid * rpt, rpt)])

    return k(msg, dst, zeros)


def _tc_edge(xi, xj, ea, w0, b0, w1, b1, w2, b2, ln_g, ln_b, gw0, gb0, gw1, gb1):
    E = xi.shape[0]
    BE = 512
    grid = E // BE

    def body(xi_ref, xj_ref, ea_ref, w0_ref, b0_ref, w1_ref, b1_ref, w2_ref,
             b2_ref, lng_ref, lnb_ref, gw0_ref, gb0_ref, gw1_ref, gb1_ref,
             eout_ref, msg_ref):
        ea_blk = ea_ref[...]
        cat = jnp.concatenate([xi_ref[...], xj_ref[...], ea_blk], axis=1)
        h = jnp.dot(cat, w0_ref[...], preferred_element_type=jnp.float32)
        h = jnp.maximum(h + b0_ref[...], 0.0)
        h = jnp.dot(h, w1_ref[...], preferred_element_type=jnp.float32)
        h = jnp.maximum(h + b1_ref[...], 0.0)
        h = jnp.dot(h, w2_ref[...], preferred_element_type=jnp.float32)
        h = h + b2_ref[...]
        m = jnp.mean(h, axis=1, keepdims=True)
        c = h - m
        v = jnp.mean(c * c, axis=1, keepdims=True)
        e_new = c * lax.rsqrt(v + 1e-5) * lng_ref[...] + lnb_ref[...]
        gh = jnp.dot(cat, gw0_ref[...], preferred_element_type=jnp.float32)
        gh = jnp.maximum(gh + gb0_ref[...], 0.0)
        z = jnp.sum(gh * gw1_ref[...], axis=1, keepdims=True) + gb1_ref[0, 0]
        gate = 1.0 / (1.0 + jnp.exp(-z))
        eout_ref[...] = ea_blk + e_new
        msg_ref[...] = gate * e_new

    blk = lambda r: pl.BlockSpec((BE, H), lambda i: (i, 0))
    full = lambda shape: pl.BlockSpec(shape, lambda i: (0,) * len(shape))
    return pl.pallas_call(
        body,
        grid=(grid,),
        in_specs=[
            blk(0), blk(0), blk(0),
            full((3 * H, H)), full((1, H)),
            full((H, H)), full((1, H)),
            full((H, H)), full((1, H)),
            full((1, H)), full((1, H)),
            full((3 * H, H)), full((1, H)),
            full((1, H)), full((1, 1)),
        ],
        out_specs=[blk(0), blk(0)],
        out_shape=[
            jax.ShapeDtypeStruct((E, H), jnp.float32),
            jax.ShapeDtypeStruct((E, H), jnp.float32),
        ],
    )(xi, xj, ea, w0, b0, w1, b1, w2, b2, ln_g, ln_b, gw0, gb0, gw1, gb1)


def _tc_node(x, a0, a1, w0, b0, w1, b1, w2, b2, ln_g, ln_b):
    N = x.shape[0]
    BN = 1000
    grid = N // BN

    def body(x_ref, a0_ref, a1_ref, w0_ref, b0_ref, w1_ref, b1_ref, w2_ref,
             b2_ref, lng_ref, lnb_ref, out_ref):
        x_blk = x_ref[...]
        aggr = a0_ref[...] + a1_ref[...]
        cat = jnp.concatenate([x_blk, aggr], axis=1)
        h = jnp.dot(cat, w0_ref[...], preferred_element_type=jnp.float32)
        h = jnp.maximum(h + b0_ref[...], 0.0)
        h = jnp.dot(h, w1_ref[...], preferred_element_type=jnp.float32)
        h = jnp.maximum(h + b1_ref[...], 0.0)
        h = jnp.dot(h, w2_ref[...], preferred_element_type=jnp.float32)
        h = h + b2_ref[...]
        m = jnp.mean(h, axis=1, keepdims=True)
        c = h - m
        v = jnp.mean(c * c, axis=1, keepdims=True)
        out_ref[...] = x_blk + (c * lax.rsqrt(v + 1e-5) * lng_ref[...]
                                + lnb_ref[...])

    blk = pl.BlockSpec((BN, H), lambda i: (i, 0))
    full = lambda shape: pl.BlockSpec(shape, lambda i: (0,) * len(shape))
    return pl.pallas_call(
        body,
        grid=(grid,),
        in_specs=[
            blk, blk, blk,
            full((2 * H, H)), full((1, H)),
            full((H, H)), full((1, H)),
            full((H, H)), full((1, H)),
            full((1, H)), full((1, H)),
        ],
        out_specs=blk,
        out_shape=jax.ShapeDtypeStruct((N, H), jnp.float32),
    )(x, a0, a1, w0, b0, w1, b1, w2, b2, ln_g, ln_b)


def kernel(x, edge_index, edge_attr, params):
    p = params
    src = edge_index[0]
    dst = edge_index[1]
    N = x.shape[0]

    xi, xj = _sc_gather(x, dst, src)

    r1 = lambda a: a.reshape(1, H)
    e_out, msg = _tc_edge(
        xi, xj, edge_attr,
        p['e_w0'], r1(p['e_b0']), p['e_w1'], r1(p['e_b1']),
        p['e_w2'], r1(p['e_b2']), r1(p['e_ln_g']), r1(p['e_ln_b']),
        p['g_w0'], r1(p['g_b0']), p['g_w1'].reshape(1, H),
        p['g_b1'].reshape(1, 1))

    zeros = jnp.zeros((NPAD, H), jnp.float32)
    parts = _sc_scatter(msg, dst, zeros)

    x_new = _tc_node(
        x, parts[0, :N], parts[1, :N],
        p['n_w0'], r1(p['n_b0']), p['n_w1'], r1(p['n_b1']),
        p['n_w2'], r1(p['n_b2']), r1(p['n_ln_g']), r1(p['n_ln_b']))

    return (x_new, e_out)
